# baseline (speedup 1.0000x reference)
import functools

import jax
import jax.numpy as jnp
from jax import lax
from jax.experimental import pallas as pl
from jax.experimental.pallas import tpu as pltpu, tpu_sc as plsc

RATE = 0.2
TOTAL_TOK = 32768
D = 2048
LANES = 16

NC, NS = 2, 16
NW = NC * NS
ROWS_PER_W = TOTAL_TOK // NW
CHUNK_ROWS = 16
CHUNK = CHUNK_ROWS * D
NCHUNKS = ROWS_PER_W // CHUNK_ROWS
WORDS_PER_W = ROWS_PER_W * D

_mesh = plsc.VectorSubcoreMesh(core_axis_name="c", subcore_axis_name="s")


@functools.partial(
    pl.kernel,
    out_type=jax.ShapeDtypeStruct((TOTAL_TOK * D,), jnp.float32),
    mesh=_mesh,
    scratch_types=[
        pltpu.VMEM((D,), jnp.float32),
        pltpu.VMEM((CHUNK,), jnp.float32),
    ],
)
def _sc_mask_mul(x_hbm, mask_hbm, out_hbm, mask_v, buf):
    wid = lax.axis_index("s") * NC + lax.axis_index("c")
    base = wid * WORDS_PER_W
    pltpu.sync_copy(mask_hbm, mask_v)

    def chunk_body(k, carry):
        off = base + k * CHUNK
        pltpu.sync_copy(x_hbm.at[pl.ds(off, CHUNK)], buf)

        def col_body(j, c2):
            m = mask_v[pl.ds(j * LANES, LANES)]
            for r in range(CHUNK_ROWS):
                sl = pl.ds(r * D + j * LANES, LANES)
                buf[sl] = buf[sl] * m
            return c2

        lax.fori_loop(0, D // LANES, col_body, 0, unroll=False)
        pltpu.sync_copy(buf, out_hbm.at[pl.ds(off, CHUNK)])
        return carry

    lax.fori_loop(0, NCHUNKS, chunk_body, 0, unroll=False)


def kernel(flat_values, row_starts):
    del row_starts
    keep = jax.random.bernoulli(jax.random.key(42), 1.0 - RATE, (D,))
    dp_mask = keep.astype(jnp.float32)
    out = _sc_mask_mul(flat_values.reshape(-1), dp_mask)
    return out.reshape(TOTAL_TOK, D)


# speedup vs baseline: 1.0594x; 1.0594x over previous
import functools

import jax
import jax.numpy as jnp
from jax import lax
from jax.experimental import pallas as pl
from jax.experimental.pallas import tpu as pltpu, tpu_sc as plsc

RATE = 0.2
TOTAL_TOK = 32768
D = 2048
LANES = 16

NC, NS = 2, 16
NW = NC * NS
ROWS_PER_W = TOTAL_TOK // NW
CHUNK_ROWS = 8
CHUNK = CHUNK_ROWS * D
NCHUNKS = ROWS_PER_W // CHUNK_ROWS
NBUF = 4
NSTEPS = NCHUNKS // NBUF
WORDS_PER_W = ROWS_PER_W * D

_mesh = plsc.VectorSubcoreMesh(core_axis_name="c", subcore_axis_name="s")


@functools.partial(
    pl.kernel,
    out_type=jax.ShapeDtypeStruct((TOTAL_TOK * D,), jnp.float32),
    mesh=_mesh,
    scratch_types=[
        pltpu.VMEM((D,), jnp.float32),
        pltpu.VMEM((NBUF, CHUNK), jnp.float32),
    ]
    + [pltpu.SemaphoreType.DMA] * (2 * NBUF),
)
def _sc_mask_mul(x_hbm, mask_hbm, out_hbm, mask_v, bufs, *sems):
    sin = sems[:NBUF]
    sout = sems[NBUF:]
    wid = lax.axis_index("s") * NC + lax.axis_index("c")
    base = wid * WORDS_PER_W
    pltpu.sync_copy(mask_hbm, mask_v)

    pltpu.async_copy(x_hbm.at[pl.ds(base, CHUNK)], bufs.at[0], sin[0])

    def step(p, carry):
        for b in range(NBUF):
            k = p * NBUF + b
            off = base + k * CHUNK
            bn = (b + 1) % NBUF

            @pl.when(jnp.logical_and(k + 1 >= NBUF, k + 1 < NCHUNKS))
            def _():
                pltpu.make_async_copy(
                    bufs.at[bn],
                    out_hbm.at[pl.ds(off + (1 - NBUF) * CHUNK, CHUNK)],
                    sout[bn],
                ).wait()

            @pl.when(k + 1 < NCHUNKS)
            def _():
                pltpu.async_copy(
                    x_hbm.at[pl.ds(off + CHUNK, CHUNK)], bufs.at[bn], sin[bn]
                )

            pltpu.make_async_copy(
                x_hbm.at[pl.ds(off, CHUNK)], bufs.at[b], sin[b]
            ).wait()

            def col_body(j, c2):
                m = mask_v[pl.ds(j * LANES, LANES)]
                for r in range(CHUNK_ROWS):
                    sl = pl.ds(r * D + j * LANES, LANES)
                    bufs[b, sl] = bufs[b, sl] * m
                return c2

            lax.fori_loop(0, D // LANES, col_body, 0, unroll=False)
            pltpu.async_copy(bufs.at[b], out_hbm.at[pl.ds(off, CHUNK)], sout[b])
        return carry

    lax.fori_loop(0, NSTEPS, step, 0, unroll=False)

    for b in range(NBUF):
        off = base + (NCHUNKS - NBUF + b) * CHUNK
        pltpu.make_async_copy(
            bufs.at[b], out_hbm.at[pl.ds(off, CHUNK)], sout[b]
        ).wait()


def kernel(flat_values, row_starts):
    del row_starts
    keep = jax.random.bernoulli(jax.random.key(42), 1.0 - RATE, (D,))
    dp_mask = keep.astype(jnp.float32)
    out = _sc_mask_mul(flat_values.reshape(-1), dp_mask)
    return out.reshape(TOTAL_TOK, D)


# speedup vs baseline: 4.0379x; 3.8115x over previous
import functools

import jax
import jax.numpy as jnp
from jax import lax
from jax.experimental import pallas as pl
from jax.experimental.pallas import tpu as pltpu, tpu_sc as plsc

RATE = 0.2
TOTAL_TOK = 32768
D = 2048
LANES = 16

NC, NS = 2, 16
NW = NC * NS
ROWS_PER_W = TOTAL_TOK // NW
CHUNK_ROWS = 8
NCHUNKS = ROWS_PER_W // CHUNK_ROWS
NBUF = 4
NSTEPS = NCHUNKS // NBUF

_mesh = plsc.VectorSubcoreMesh(core_axis_name="c", subcore_axis_name="s")


@functools.partial(
    pl.kernel,
    out_type=jax.ShapeDtypeStruct((TOTAL_TOK, D), jnp.float32),
    mesh=_mesh,
    scratch_types=[
        pltpu.VMEM((D,), jnp.float32),
        pltpu.VMEM((NBUF, CHUNK_ROWS, D), jnp.float32),
    ]
    + [pltpu.SemaphoreType.DMA] * (2 * NBUF),
    compiler_params=pltpu.CompilerParams(use_tc_tiling_on_sc=True),
)
def _sc_mask_mul(x_hbm, mask_hbm, out_hbm, mask_v, bufs, *sems):
    sin = sems[:NBUF]
    sout = sems[NBUF:]
    wid = lax.axis_index("s") * NC + lax.axis_index("c")
    base = wid * ROWS_PER_W
    pltpu.sync_copy(mask_hbm, mask_v)

    pltpu.async_copy(
        x_hbm.at[pl.ds(base, CHUNK_ROWS), :], bufs.at[0], sin[0]
    )

    def step(p, carry):
        for b in range(NBUF):
            k = p * NBUF + b
            row0 = base + k * CHUNK_ROWS
            bn = (b + 1) % NBUF

            @pl.when(jnp.logical_and(k + 1 >= NBUF, k + 1 < NCHUNKS))
            def _():
                pltpu.make_async_copy(
                    bufs.at[bn],
                    out_hbm.at[
                        pl.ds(row0 + (1 - NBUF) * CHUNK_ROWS, CHUNK_ROWS), :
                    ],
                    sout[bn],
                ).wait()

            @pl.when(k + 1 < NCHUNKS)
            def _():
                pltpu.async_copy(
                    x_hbm.at[pl.ds(row0 + CHUNK_ROWS, CHUNK_ROWS), :],
                    bufs.at[bn],
                    sin[bn],
                )

            pltpu.make_async_copy(
                x_hbm.at[pl.ds(row0, CHUNK_ROWS), :], bufs.at[b], sin[b]
            ).wait()

            def col_body(t, c2):
                c0 = t * 128
                ms = [
                    mask_v[pl.ds(c0 + v * LANES, LANES)] for v in range(8)
                ]
                for r in range(CHUNK_ROWS):
                    for v in range(8):
                        sl = pl.ds(c0 + v * LANES, LANES)
                        bufs[b, r, sl] = bufs[b, r, sl] * ms[v]
                return c2

            lax.fori_loop(0, D // 128, col_body, 0, unroll=False)
            pltpu.async_copy(
                bufs.at[b], out_hbm.at[pl.ds(row0, CHUNK_ROWS), :], sout[b]
            )
        return carry

    lax.fori_loop(0, NSTEPS, step, 0, unroll=False)

    for b in range(NBUF):
        row0 = base + (NCHUNKS - NBUF + b) * CHUNK_ROWS
        pltpu.make_async_copy(
            bufs.at[b], out_hbm.at[pl.ds(row0, CHUNK_ROWS), :], sout[b]
        ).wait()


def kernel(flat_values, row_starts):
    del row_starts
    keep = jax.random.bernoulli(jax.random.key(42), 1.0 - RATE, (D,))
    dp_mask = keep.astype(jnp.float32)
    return _sc_mask_mul(flat_values, dp_mask)
